# baseline (speedup 1.0000x reference)
import functools

import jax
import jax.numpy as jnp
import numpy as np
from jax import lax
from jax.experimental import pallas as pl
from jax.experimental.pallas import tpu as pltpu
from jax.experimental.pallas import tpu_sc as plsc

B = 16384
C = 1000
EPS = 1e-8
SIGN = np.int32(-2147483648)
LN2 = float(np.log(np.float32(2.0)))

BSC = 4096
BT = B - BSC

CB = 40
NB = C // CB
SUB = 8


def _w(m, base):
    df = m + lax.convert_element_type(base, jnp.float32)
    na = lax.bitcast_convert_type(
        lax.bitcast_convert_type(df, jnp.int32) | SIGN, jnp.float32
    )
    return jnp.exp2(na)


def _tc_body(y_ref, xt_ref, o_ref, acc_ref, m_ref):
    j = pl.program_id(0)

    @pl.when(j == 0)
    def _():
        iota = lax.broadcasted_iota(jnp.int32, (SUB, BT), 0)
        m_ref[...] = (iota - y_ref[...]).astype(jnp.float32)
        acc_ref[...] = jnp.zeros_like(acc_ref)

    m = m_ref[...]
    acc_ref[...] += sum(
        _w(m, j * CB + k * SUB) * xt_ref[pl.ds(k * SUB, SUB), :]
        for k in range(CB // SUB)
    )

    @pl.when(j == NB - 1)
    def _():
        s = jnp.sum(acc_ref[...], axis=0, keepdims=True)
        o_ref[0, 0] = jnp.sum(-jnp.log(s + EPS))


_tc_call = pl.pallas_call(
    _tc_body,
    grid=(NB,),
    in_specs=[
        pl.BlockSpec((1, BT), lambda j: (0, 0)),
        pl.BlockSpec((CB, BT), lambda j: (j, 0)),
    ],
    out_specs=pl.BlockSpec((1, 1), lambda j: (0, 0), memory_space=pltpu.SMEM),
    out_shape=jax.ShapeDtypeStruct((1, 1), jnp.float32),
    scratch_shapes=[
        pltpu.VMEM((SUB, BT), jnp.float32),
        pltpu.VMEM((SUB, BT), jnp.float32),
    ],
)

CCB = 40
NCC = C // CCB
LW = 128
NL = LW // 16
NC_SC = 2


def _sc_body(xt_hbm, y_hbm, out_hbm, y_v, yf_v, buf0, buf1, s_v, sem0, sem1):
    wid = lax.axis_index("s") * NC_SC + lax.axis_index("c")
    c0 = BT + wid * LW
    pltpu.sync_copy(y_hbm.at[pl.ds(c0, LW)], y_v)

    for l in range(NL):
        yv = y_v[pl.ds(l * 16, 16)]
        yf_v[pl.ds(l * 16, 16)] = yv.astype(jnp.float32) * (-LN2)

    def src(cc):
        return xt_hbm.at[pl.ds(cc * CCB, CCB), pl.ds(c0, LW)]

    pltpu.async_copy(src(0), buf0, sem0)
    pltpu.async_copy(src(1), buf1, sem1)

    def chunk(cc, buf, accs):
        base_f = lax.convert_element_type(cc * CCB, jnp.float32) * LN2
        yfs = [yf_v[pl.ds(l * 16, 16)] for l in range(NL)]
        out = list(accs)
        for r in range(CCB):
            cfl = base_f + float(r * LN2)
            for l in range(NL):
                v = buf[r, pl.ds(l * 16, 16)]
                t = yfs[l] + cfl
                na = lax.bitcast_convert_type(
                    lax.bitcast_convert_type(t, jnp.int32) | SIGN, jnp.float32
                )
                out[l] = out[l] + jnp.exp(na) * v
        return tuple(out)

    def body(g, accs):
        cc0 = 2 * g

        pltpu.make_async_copy(src(0), buf0, sem0).wait()

        @pl.when(cc0 + 2 < NCC)
        def _():
            pltpu.async_copy(src(cc0 + 2), buf0, sem0)

        accs = chunk(cc0, buf0, accs)

        pltpu.make_async_copy(src(1), buf1, sem1).wait()

        @pl.when(cc0 + 3 < NCC)
        def _():
            pltpu.async_copy(src(cc0 + 3), buf1, sem1)

        accs = chunk(cc0 + 1, buf1, accs)
        return accs

    accs0 = tuple(jnp.zeros((16,), jnp.float32) for _ in range(NL))
    accs = lax.fori_loop(0, NCC // 2, body, accs0)

    pltpu.make_async_copy(src(0), buf0, sem0).wait()
    accs = chunk(NCC - 1, buf0, accs)

    for l in range(NL):
        s_v[pl.ds(l * 16, 16)] = accs[l]
    pltpu.sync_copy(s_v, out_hbm.at[pl.ds(wid * LW, LW)])


_sc_call = functools.partial(
    pl.kernel,
    out_type=jax.ShapeDtypeStruct((BSC,), jnp.float32),
    mesh=plsc.VectorSubcoreMesh(core_axis_name="c", subcore_axis_name="s"),
    scratch_types=[
        pltpu.VMEM((LW,), jnp.int32),
        pltpu.VMEM((LW,), jnp.float32),
        pltpu.VMEM((CCB, LW), jnp.float32),
        pltpu.VMEM((CCB, LW), jnp.float32),
        pltpu.VMEM((LW,), jnp.float32),
        pltpu.SemaphoreType.DMA,
        pltpu.SemaphoreType.DMA,
    ],
    compiler_params=pltpu.CompilerParams(
        needs_layout_passes=False, use_tc_tiling_on_sc=True
    ),
)(_sc_body)


def _comb_body(ssc_ref, ptc_ref, o_ref):
    t = jnp.sum(-jnp.log(ssc_ref[...] + EPS))
    o_ref[0, 0] = (t + ptc_ref[0, 0]) * (1.0 / B)


_comb_call = pl.pallas_call(
    _comb_body,
    in_specs=[
        pl.BlockSpec(memory_space=pltpu.VMEM),
        pl.BlockSpec(memory_space=pltpu.SMEM),
    ],
    out_specs=pl.BlockSpec(memory_space=pltpu.SMEM),
    out_shape=jax.ShapeDtypeStruct((1, 1), jnp.float32),
)


def kernel(x, y):
    y32 = y.astype(jnp.int32)
    xt = x.T
    s_sc = _sc_call(xt, y32)
    p_tc = _tc_call(y32[:BT].reshape(1, BT), xt)
    return _comb_call(s_sc.reshape(32, 128), p_tc)[0, 0]


# speedup vs baseline: 2.0595x; 2.0595x over previous
import jax
import jax.numpy as jnp
import numpy as np
from jax import lax
from jax.experimental import pallas as pl
from jax.experimental.pallas import tpu as pltpu

B = 16384
C = 1000
EPS = 1e-8
CB = 200
NB = C // CB
SUB = 8
SIGN = np.int32(-2147483648)


def _w(m, base):
    df = m + lax.convert_element_type(base, jnp.float32)
    na = lax.bitcast_convert_type(
        lax.bitcast_convert_type(df, jnp.int32) | SIGN, jnp.float32
    )
    return jnp.exp2(na)


def _body(y_ref, xt_ref, o_ref, acc_ref, m_ref):
    j = pl.program_id(0)

    @pl.when(j == 0)
    def _():
        iota = lax.broadcasted_iota(jnp.int32, (SUB, B), 0)
        m_ref[...] = (iota - y_ref[...]).astype(jnp.float32)
        acc_ref[...] = jnp.zeros_like(acc_ref)

    m = m_ref[...]
    acc_ref[...] += sum(
        _w(m, j * CB + k * SUB) * xt_ref[pl.ds(k * SUB, SUB), :]
        for k in range(CB // SUB)
    )

    @pl.when(j == NB - 1)
    def _():
        s = jnp.sum(acc_ref[...], axis=0, keepdims=True)
        o_ref[0, 0] = jnp.sum(-jnp.log(s + EPS)) * (1.0 / B)


_call = pl.pallas_call(
    _body,
    grid=(NB,),
    in_specs=[
        pl.BlockSpec((1, B), lambda j: (0, 0)),
        pl.BlockSpec((CB, B), lambda j: (j, 0)),
    ],
    out_specs=pl.BlockSpec((1, 1), lambda j: (0, 0), memory_space=pltpu.SMEM),
    out_shape=jax.ShapeDtypeStruct((1, 1), jnp.float32),
    scratch_shapes=[
        pltpu.VMEM((SUB, B), jnp.float32),
        pltpu.VMEM((SUB, B), jnp.float32),
    ],
)


def kernel(x, y):
    y2 = y.astype(jnp.int32).reshape(1, B)
    return _call(y2, x.T)[0, 0]
